# Initial kernel scaffold; baseline (speedup 1.0000x reference)
import jax
import jax.numpy as jnp
from jax.experimental import pallas as pl


def kernel(x, edge_index, batch, W1a, b1a, W1b, b1b, W2a, b2a, W2b, b2b, eps1, eps2, Wc1, bc1, g, beta, Wc2, bc2):
    raise NotImplementedError("write your pallas kernel here")


# speedup vs baseline: 3.7377x; 3.7377x over previous
import dataclasses
import functools

import jax
import jax.numpy as jnp
from jax import lax
from jax.experimental import pallas as pl
from jax.experimental.pallas import tpu as pltpu
from jax.experimental.pallas import tpu_sc as plsc

N = 100000
E = 1600000
B = 512
NPAD = 102400
EPT = E // 16
CHUNK = 2000
KB = 128
SEG = 544

_mesh = plsc.VectorSubcoreMesh(core_axis_name="c", subcore_axis_name="s")


def _sc_params():
    cp = pltpu.CompilerParams(use_tc_tiling_on_sc=False)
    if "needs_layout_passes" in pltpu.CompilerParams.__dataclass_fields__:
        cp = dataclasses.replace(cp, needs_layout_passes=False)
    return cp


def _make_agg(F, R, SWEEPS):
    RPT = R // 16
    ZR = 160
    NCH = EPT // CHUNK
    NIT = CHUNK // 16

    def body(feat, src, dst, zsrc, agg, schunk, dchunk, srcbuf, dstbuf,
             dstbatch, rows_v, zbuf, acc):
        cid = lax.axis_index("c")
        sid = lax.axis_index("s")
        e0 = sid * EPT
        pltpu.sync_copy(zsrc, zbuf)

        def drain(cnt):
            for k in range(KB // 16):
                dstbatch[pl.ds(k * 16, 16)] = dstbuf[pl.ds(k * 16, 16)]
            pltpu.sync_copy(feat.at[srcbuf.at[pl.ds(0, KB)]], rows_v)
            pltpu.sync_copy(rows_v, acc.at[dstbatch], add=True)
            srcbuf[pl.ds(0, 16)] = srcbuf[pl.ds(KB, 16)]
            dstbuf[pl.ds(0, 16)] = dstbuf[pl.ds(KB, 16)]
            return cnt - KB

        @pl.loop(0, SWEEPS)
        def sweep_body(sw):
            lo = (2 * sw + cid) * R
            hi = lo + R
            @pl.loop(0, RPT // ZR)
            def _(z):
                pltpu.sync_copy(zbuf, acc.at[pl.ds(sid * RPT + z * ZR, ZR)])
            plsc.subcore_barrier()

            def chunk_body(ch, cnt):
                pltpu.sync_copy(src.at[pl.ds(e0 + ch * CHUNK, CHUNK)], schunk)
                pltpu.sync_copy(dst.at[pl.ds(e0 + ch * CHUNK, CHUNK)], dchunk)

                def step(i, cnt):
                    dv = dchunk[pl.ds(i * 16, 16)]
                    sv = schunk[pl.ds(i * 16, 16)]
                    m = (dv >= lo) & (dv < hi)
                    plsc.store_compressed(srcbuf.at[pl.ds(cnt, 16)], sv, mask=m)
                    plsc.store_compressed(dstbuf.at[pl.ds(cnt, 16)], dv - lo, mask=m)
                    cnt = cnt + jnp.max(plsc.all_reduce_population_count(m))
                    return lax.cond(cnt >= KB, drain, lambda c: c, cnt)

                return lax.fori_loop(0, NIT, step, cnt)

            cnt = lax.fori_loop(0, NCH, chunk_body, jnp.int32(0))
            zi = jnp.zeros((16,), jnp.int32)
            ri = jnp.full((16,), R, jnp.int32)
            for k in range(9):
                srcbuf[pl.ds(cnt + k * 16, 16)] = zi
                dstbuf[pl.ds(cnt + k * 16, 16)] = ri
            drain(cnt)
            plsc.subcore_barrier()
            pltpu.sync_copy(acc.at[pl.ds(sid * RPT, RPT)],
                            agg.at[pl.ds(lo + sid * RPT, RPT)])

    return pl.kernel(
        body,
        out_type=jax.ShapeDtypeStruct((NPAD, F), jnp.float32),
        mesh=_mesh,
        compiler_params=_sc_params(),
        scratch_types=[
            pltpu.VMEM((CHUNK,), jnp.int32),
            pltpu.VMEM((CHUNK,), jnp.int32),
            pltpu.VMEM((304,), jnp.int32),
            pltpu.VMEM((304,), jnp.int32),
            pltpu.VMEM((KB,), jnp.int32),
            pltpu.VMEM((KB, F), jnp.float32),
            pltpu.VMEM((ZR, F), jnp.float32),
            pltpu.VMEM_SHARED((R + 8, F), jnp.float32),
        ],
    )


def _make_pool(F):
    RPC = NPAD // 2
    RPT = RPC // 16
    NCH = RPT // KB

    def body(feat, batchp, zsrc, out, fchunk, bbatch, zbuf, acc):
        cid = lax.axis_index("c")
        sid = lax.axis_index("s")
        pltpu.sync_copy(zsrc, zbuf)
        pltpu.sync_copy(zbuf, acc.at[pl.ds(sid * (SEG // 16), SEG // 16)])
        plsc.subcore_barrier()
        row0 = cid * RPC + sid * RPT

        @pl.loop(0, NCH)
        def _(ch):
            pltpu.sync_copy(feat.at[pl.ds(row0 + ch * KB, KB)], fchunk)
            pltpu.sync_copy(batchp.at[pl.ds(row0 + ch * KB, KB)], bbatch)
            pltpu.sync_copy(fchunk, acc.at[bbatch], add=True)

        plsc.subcore_barrier()
        pltpu.sync_copy(acc.at[pl.ds(sid * 32, 32)],
                        out.at[pl.ds(cid * 512 + sid * 32, 32)])

    return pl.kernel(
        body,
        out_type=jax.ShapeDtypeStruct((1024, F), jnp.float32),
        mesh=_mesh,
        compiler_params=_sc_params(),
        scratch_types=[
            pltpu.VMEM((KB, F), jnp.float32),
            pltpu.VMEM((KB,), jnp.int32),
            pltpu.VMEM((SEG // 16, F), jnp.float32),
            pltpu.VMEM_SHARED((SEG, F), jnp.float32),
        ],
    )


def _mlp_body(e_ref, h_ref, a_ref, wa_ref, ba_ref, wb_ref, bb_ref, o_ref):
    t = (1.0 + e_ref[0, 0]) * h_ref[...] + a_ref[...]
    u = jnp.maximum(
        jnp.dot(t, wa_ref[...], preferred_element_type=jnp.float32)
        + ba_ref[...], 0.0)
    o_ref[...] = (jnp.dot(u, wb_ref[...], preferred_element_type=jnp.float32)
                  + bb_ref[...])


def _mlp(eps, h, agg, Wa, ba, Wb, bb):
    F = h.shape[1]
    TR = 512
    grid = (NPAD // TR,)
    return pl.pallas_call(
        _mlp_body,
        grid=grid,
        in_specs=[
            pl.BlockSpec(memory_space=pltpu.SMEM),
            pl.BlockSpec((TR, F), lambda i: (i, 0)),
            pl.BlockSpec((TR, F), lambda i: (i, 0)),
            pl.BlockSpec((F, 128), lambda i: (0, 0)),
            pl.BlockSpec((1, 128), lambda i: (0, 0)),
            pl.BlockSpec((128, 128), lambda i: (0, 0)),
            pl.BlockSpec((1, 128), lambda i: (0, 0)),
        ],
        out_specs=pl.BlockSpec((TR, 128), lambda i: (i, 0)),
        out_shape=jax.ShapeDtypeStruct((NPAD, 128), jnp.float32),
    )(eps, h, agg, Wa, ba, Wb, bb)


def _head_body(p48_ref, pa_ref, pb_ref, w1_ref, b1_ref, g_ref, be_ref,
               w2_ref, b2_ref, o_ref):
    pool = jnp.concatenate(
        [p48_ref[:512] + p48_ref[512:],
         pa_ref[:512] + pa_ref[512:],
         pb_ref[:512] + pb_ref[512:]], axis=1)
    h = jnp.dot(pool, w1_ref[...], preferred_element_type=jnp.float32) + b1_ref[...]
    mu = jnp.mean(h, axis=-1, keepdims=True)
    var = jnp.mean((h - mu) ** 2, axis=-1, keepdims=True)
    h = (h - mu) * jax.lax.rsqrt(var + 1e-5) * g_ref[...] + be_ref[...]
    h = jnp.where(h >= 0, h, 0.01 * h)
    o_ref[...] = (jnp.dot(h, w2_ref[...], preferred_element_type=jnp.float32)
                  + b2_ref[...])


def _head(p48, pa, pb, Wc1p, bc1, g, beta, Wc2, bc2):
    return pl.pallas_call(
        _head_body,
        out_shape=jax.ShapeDtypeStruct((B, Wc2.shape[1]), jnp.float32),
    )(p48, pa, pb, Wc1p, bc1, g, beta, Wc2, bc2)


_agg48 = _make_agg(48, 25600, 2)
_agg128 = _make_agg(128, 10240, 5)
_pool48 = _make_pool(48)
_pool128 = _make_pool(128)


def kernel(x, edge_index, batch, W1a, b1a, W1b, b1b, W2a, b2a, W2b, b2b,
           eps1, eps2, Wc1, bc1, g, beta, Wc2, bc2):
    f32 = jnp.float32
    x48 = jnp.pad(x, ((0, NPAD - N), (0, 2)))
    src = edge_index[0]
    dst = edge_index[1]
    batchp = jnp.concatenate(
        [batch, jnp.full((NPAD - N,), B, jnp.int32)])
    z48 = jnp.zeros((160, 48), f32)
    z128 = jnp.zeros((160, 128), f32)
    zp48 = jnp.zeros((SEG // 16, 48), f32)
    zp128 = jnp.zeros((SEG // 16, 128), f32)

    W1ap = jnp.pad(W1a, ((0, 2), (0, 0)))
    Wc1p = jnp.concatenate([Wc1[:46], jnp.zeros((2, 64), f32), Wc1[46:]])

    e1 = eps1.reshape(1, 1)
    e2 = eps2.reshape(1, 1)

    agg1 = _agg48(x48, src, dst, z48)
    h1 = _mlp(e1, x48, agg1, W1ap, b1a.reshape(1, -1), W1b, b1b.reshape(1, -1))
    agg2 = _agg128(h1, src, dst, z128)
    h2 = _mlp(e2, h1, agg2, W2a, b2a.reshape(1, -1), W2b, b2b.reshape(1, -1))

    p48 = _pool48(x48, batchp, zp48)
    pa = _pool128(h1, batchp, zp128)
    pb = _pool128(h2, batchp, zp128)

    return _head(p48, pa, pb, Wc1p, bc1.reshape(1, -1), g.reshape(1, -1),
                 beta.reshape(1, -1), Wc2, bc2.reshape(1, -1))
